# Initial kernel scaffold; baseline (speedup 1.0000x reference)
import jax
import jax.numpy as jnp
from jax.experimental import pallas as pl


def kernel(words, speakers, emotions, graphs, spans, params):
    raise NotImplementedError("write your pallas kernel here")


# speedup vs baseline: 4.6083x; 4.6083x over previous
import functools

import jax
import jax.numpy as jnp
from jax import lax
from jax.experimental import pallas as pl
from jax.experimental.pallas import tpu as pltpu
from jax.experimental.pallas import tpu_sc as plsc

B, C, U, F = 4, 24, 16, 4
E, H, DS, DE = 128, 128, 64, 64
VW, VS, VE = 100000, 10, 8
SH = E // 2
BC = B * C
NARC = B * C * C
NIDX = B * C * U * F


def _sc_gather(table, idx):
    info = plsc.get_sparse_core_info()
    nc, ns = info.num_cores, info.num_subcores
    nw = nc * ns
    bpw = NIDX // nw
    mesh = plsc.VectorSubcoreMesh(core_axis_name="c", subcore_axis_name="s")

    @functools.partial(
        pl.kernel,
        mesh=mesh,
        out_type=jax.ShapeDtypeStruct((NIDX, E), jnp.float32),
        scratch_types=[
            pltpu.VMEM((bpw,), jnp.int32),
            pltpu.VMEM((bpw, E), jnp.float32),
            pltpu.SemaphoreType.DMA,
        ],
    )
    def k(table_hbm, idx_hbm, out_hbm, idx_v, rows_v, sem):
        wid = lax.axis_index("s") * nc + lax.axis_index("c")
        base = wid * bpw
        pltpu.sync_copy(idx_hbm.at[pl.ds(base, bpw)], idx_v)
        pltpu.async_copy(table_hbm.at[idx_v], rows_v, sem).wait()
        pltpu.sync_copy(rows_v, out_hbm.at[pl.ds(base, bpw)])

    return k(table, idx)


def _lstm_dir(x_steps, A, R, b, reverse):
    n = x_steps[0].shape[0]
    h4 = R.shape[1]
    hh = h4 // 4
    h = jnp.zeros((n, hh), jnp.float32)
    c = jnp.zeros((n, hh), jnp.float32)
    order = range(len(x_steps) - 1, -1, -1) if reverse else range(len(x_steps))
    for t in order:
        gates = (
            jax.lax.dot_general(x_steps[t], A, (((1,), (0,)), ((), ())),
                                preferred_element_type=jnp.float32)
            + jax.lax.dot_general(h, R, (((1,), (0,)), ((), ())),
                                  preferred_element_type=jnp.float32)
            + b
        )
        gi = jax.nn.sigmoid(gates[:, 0:hh])
        gf = jax.nn.sigmoid(gates[:, hh:2 * hh])
        gg = jnp.tanh(gates[:, 2 * hh:3 * hh])
        go = jax.nn.sigmoid(gates[:, 3 * hh:4 * hh])
        c = gf * c + gi * gg
        h = go * jnp.tanh(c)
    return h


def _encode_body(g_ref, spk_ids_ref, em_ids_ref,
                 utA_f_ref, utR_f_ref, utb_f_ref,
                 utA_b_ref, utR_b_ref, utb_b_ref,
                 spk_tab_ref, em_tab_ref,
                 wc_ref, wcb_ref, we_ref, web_ref,
                 emc_ref, emcb_ref, eme_ref, emeb_ref,
                 wut_ref, wem_ref,
                 spWw_f_ref, spWe_f_ref, spb_f_ref,
                 spWw_b_ref, spWe_b_ref, spb_b_ref,
                 sut_ref, sem_ref, xwf_ref, xwb_ref, ebf_ref, ebb_ref):
    g = g_ref[...]
    we = (g[0] + g[1] + g[2] + g[3]) * 0.25
    we3 = we.reshape(U, BC, E)
    x_steps = [we3[t] for t in range(U)]

    spWw_f = spWw_f_ref[...]
    spWw_b = spWw_b_ref[...]
    for t in range(U):
        xwf_ref[t] = jax.lax.dot_general(
            x_steps[t], spWw_f, (((1,), (0,)), ((), ())),
            preferred_element_type=jnp.float32)
        xwb_ref[t] = jax.lax.dot_general(
            x_steps[t], spWw_b, (((1,), (0,)), ((), ())),
            preferred_element_type=jnp.float32)

    hT_f = _lstm_dir(x_steps, utA_f_ref[...], utR_f_ref[...], utb_f_ref[...],
                     reverse=False)
    hT_b = _lstm_dir(x_steps, utA_b_ref[...], utR_b_ref[...], utb_b_ref[...],
                     reverse=True)

    spk_oh = (spk_ids_ref[...] ==
              jax.lax.broadcasted_iota(jnp.int32, (BC, VS), 1)
              ).astype(jnp.float32)
    spk = jax.lax.dot_general(spk_oh, spk_tab_ref[...], (((1,), (0,)), ((), ())),
                              preferred_element_type=jnp.float32)
    em_oh = (em_ids_ref[...] ==
             jax.lax.broadcasted_iota(jnp.int32, (BC, VE), 1)
             ).astype(jnp.float32)
    em_e = jax.lax.dot_general(em_oh, em_tab_ref[...], (((1,), (0,)), ((), ())),
                               preferred_element_type=jnp.float32)

    ebf_ref[...] = jax.lax.dot_general(
        em_e, spWe_f_ref[...], (((1,), (0,)), ((), ())),
        preferred_element_type=jnp.float32) + spb_f_ref[...]
    ebb_ref[...] = jax.lax.dot_general(
        em_e, spWe_b_ref[...], (((1,), (0,)), ((), ())),
        preferred_element_type=jnp.float32) + spb_b_ref[...]

    ut = jnp.concatenate([hT_f, hT_b, spk], axis=-1)

    def ffn(wref, bref):
        y = jax.lax.dot_general(ut, wref[...], (((1,), (0,)), ((), ())),
                                preferred_element_type=jnp.float32) + bref[...]
        return jnp.where(y >= 0, y, 0.1 * y)

    ut_cause = ffn(wc_ref, wcb_ref)
    ut_effect = ffn(we_ref, web_ref)
    em_cause = ffn(emc_ref, emcb_ref)
    em_effect = ffn(eme_ref, emeb_ref)

    ones = jnp.ones((C, 1), jnp.float32)
    wut = wut_ref[...]
    wem = wem_ref[...]
    for bb in range(B):
        r0 = bb * C
        xe = jnp.concatenate([ut_effect[r0:r0 + C], ones], axis=-1)
        yc = ut_cause[r0:r0 + C]
        t1 = jax.lax.dot_general(xe, wut, (((1,), (0,)), ((), ())),
                                 preferred_element_type=jnp.float32)
        sut_ref[bb] = jax.lax.dot_general(t1, yc, (((1,), (1,)), ((), ())),
                                          preferred_element_type=jnp.float32)
        xem = jnp.concatenate([em_effect[r0:r0 + C], ones], axis=-1)
        yem = jnp.concatenate([em_cause[r0:r0 + C], ones], axis=-1)
        for o in range(VE):
            t2 = jax.lax.dot_general(xem, wem[o], (((1,), (0,)), ((), ())),
                                     preferred_element_type=jnp.float32)
            sem_ref[bb, o] = jax.lax.dot_general(
                t2, yem, (((1,), (1,)), ((), ())),
                preferred_element_type=jnp.float32)


def _encode_call(g, spk_ids, em_ids, p):
    out_shapes = [
        jax.ShapeDtypeStruct((B, C, C), jnp.float32),
        jax.ShapeDtypeStruct((B, VE, C, C), jnp.float32),
        jax.ShapeDtypeStruct((U, BC, 4 * SH), jnp.float32),
        jax.ShapeDtypeStruct((U, BC, 4 * SH), jnp.float32),
        jax.ShapeDtypeStruct((BC, 4 * SH), jnp.float32),
        jax.ShapeDtypeStruct((BC, 4 * SH), jnp.float32),
    ]
    args = [
        g.reshape(F, U * BC, E),
        spk_ids, em_ids,
        p['ut_Wih_f'].T, p['ut_Whh_f'].T, p['ut_b_f'].reshape(1, -1),
        p['ut_Wih_b'].T, p['ut_Whh_b'].T, p['ut_b_b'].reshape(1, -1),
        p['spk_table'], p['em_table'],
        p['ut_cause_W'].T, p['ut_cause_b'].reshape(1, -1),
        p['ut_effect_W'].T, p['ut_effect_b'].reshape(1, -1),
        p['em_cause_W'].T, p['em_cause_b'].reshape(1, -1),
        p['em_effect_W'].T, p['em_effect_b'].reshape(1, -1),
        p['W_ut'][0], p['W_em'],
        p['sp_Wih_f'][:, :E].T, p['sp_Wih_f'][:, E:].T,
        p['sp_b_f'].reshape(1, -1),
        p['sp_Wih_b'][:, :E].T, p['sp_Wih_b'][:, E:].T,
        p['sp_b_b'].reshape(1, -1),
    ]
    return pl.pallas_call(_encode_body, out_shape=out_shapes)(*args)


def _span_body(xwf_ref, xwb_ref, ebf_ref, ebb_ref,
               spR_f_ref, spR_b_ref, wf_ref, wb_ref, ob_ref,
               sut_ref, g_ref, out_ref):
    h4 = 4 * SH

    def expand_cau(x):
        x4 = x.reshape(B, 1, C, h4)
        return jnp.broadcast_to(x4, (B, C, C, h4)).reshape(NARC, h4)

    def expand_eff(x):
        x4 = x.reshape(B, C, 1, h4)
        return jnp.broadcast_to(x4, (B, C, C, h4)).reshape(NARC, h4)

    eb_f = expand_eff(ebf_ref[...])
    eb_b = expand_eff(ebb_ref[...])
    spR_f = spR_f_ref[...]
    spR_b = spR_b_ref[...]
    wf = wf_ref[...]
    wb = wb_ref[...]

    h = jnp.zeros((NARC, SH), jnp.float32)
    c = jnp.zeros((NARC, SH), jnp.float32)
    pf = [None] * U
    for t in range(U):
        gates = (expand_cau(xwf_ref[t]) + eb_f
                 + jax.lax.dot_general(h, spR_f, (((1,), (0,)), ((), ())),
                                       preferred_element_type=jnp.float32))
        gi = jax.nn.sigmoid(gates[:, 0:SH])
        gf = jax.nn.sigmoid(gates[:, SH:2 * SH])
        gg = jnp.tanh(gates[:, 2 * SH:3 * SH])
        go = jax.nn.sigmoid(gates[:, 3 * SH:4 * SH])
        c = gf * c + gi * gg
        h = go * jnp.tanh(c)
        pf[t] = jax.lax.dot_general(h, wf, (((1,), (0,)), ((), ())),
                                    preferred_element_type=jnp.float32)
    h = jnp.zeros((NARC, SH), jnp.float32)
    c = jnp.zeros((NARC, SH), jnp.float32)
    for s in range(U):
        t = U - 1 - s
        gates = (expand_cau(xwb_ref[t]) + eb_b
                 + jax.lax.dot_general(h, spR_b, (((1,), (0,)), ((), ())),
                                       preferred_element_type=jnp.float32))
        gi = jax.nn.sigmoid(gates[:, 0:SH])
        gf = jax.nn.sigmoid(gates[:, SH:2 * SH])
        gg = jnp.tanh(gates[:, 2 * SH:3 * SH])
        go = jax.nn.sigmoid(gates[:, 3 * SH:4 * SH])
        c = gf * c + gi * gg
        h = go * jnp.tanh(c)
        pf[t] = pf[t] + jax.lax.dot_general(h, wb, (((1,), (0,)), ((), ())),
                                            preferred_element_type=jnp.float32)
    logit = jnp.concatenate(pf, axis=1) + ob_ref[...]
    preds = jax.nn.sigmoid(logit)
    mask = (g_ref[...] != 0) | (sut_ref[...] > 0.0)
    out_ref[...] = jnp.where(jnp.broadcast_to(mask, (NARC, U)),
                             preds, jnp.zeros((NARC, U), jnp.float32))


def _span_call(xw_f, xw_b, eb_f, eb_b, sut_col, g_col, p):
    out_shape = jax.ShapeDtypeStruct((NARC, U), jnp.float32)
    args = [
        xw_f, xw_b, eb_f, eb_b,
        p['sp_Whh_f'].T, p['sp_Whh_b'].T,
        p['sp_out_W'][:, :SH].T, p['sp_out_W'][:, SH:].T,
        p['sp_out_b'].reshape(1, 1),
        sut_col, g_col,
    ]
    return pl.pallas_call(_span_body, out_shape=out_shape)(*args)


def kernel(words, speakers, emotions, graphs, spans, params):
    del spans
    idx = words.astype(jnp.int32).transpose(3, 2, 0, 1).reshape(-1)
    g = _sc_gather(params['word_table'], idx)

    spk_ids = speakers.astype(jnp.int32).reshape(BC, 1)
    em_ids = emotions.astype(jnp.int32).reshape(BC, 1)
    s_ut, s_em_k, xw_f, xw_b, eb_f, eb_b = _encode_call(
        g, spk_ids, em_ids, params)

    sut_col = s_ut.reshape(NARC, 1)
    g_col = graphs.astype(jnp.int32).reshape(NARC, 1)
    sp = _span_call(xw_f, xw_b, eb_f, eb_b, sut_col, g_col, params)

    s_em = jnp.transpose(s_em_k, (0, 2, 3, 1))
    s_span = sp.reshape(B, C, C, U)
    return (s_ut, s_em, s_span)
